# baseline (speedup 1.0000x reference)
import functools

import jax
import jax.numpy as jnp
from jax.experimental import pallas as pl
from jax.experimental.pallas import tpu as pltpu


def _ffn_kernel(x_ref, cond_ref, w1_ref, b1_ref, w2_ref, b2_ref, out_ref):
    f = pl.program_id(1)
    a = x_ref[0] + cond_ref[0]
    h = jax.lax.dot_general(
        a, w1_ref[0],
        dimension_numbers=(((1,), (1,)), ((), ())),
        preferred_element_type=jnp.float32,
    )
    h = jax.nn.gelu(h + b1_ref[0])
    o = jax.lax.dot_general(
        h.astype(jnp.bfloat16), w2_ref[0],
        dimension_numbers=(((1,), (1,)), ((), ())),
        preferred_element_type=jnp.float32,
    )

    @pl.when(f == 0)
    def _init():
        out_ref[0] = o + b2_ref[0]

    @pl.when(f != 0)
    def _acc():
        out_ref[0] += o


def kernel(x, cond, W1, b1, W2, b2):
    E, T, D = x.shape
    F = W1.shape[1]
    BF = min(512, F)
    nf = F // BF

    xc = x.astype(jnp.bfloat16)
    cc = cond.astype(jnp.bfloat16)
    W1c = W1.astype(jnp.bfloat16)
    W2c = W2.astype(jnp.bfloat16)

    b1r = b1.reshape(E * nf, 1, BF)
    b2r = b2.reshape(E, 1, D)

    grid = (E, nf)
    return pl.pallas_call(
        _ffn_kernel,
        grid=grid,
        in_specs=[
            pl.BlockSpec((1, T, D), lambda e, f: (e, 0, 0)),
            pl.BlockSpec((1, T, D), lambda e, f: (e, 0, 0)),
            pl.BlockSpec((1, BF, D), lambda e, f: (e, f, 0)),
            pl.BlockSpec((1, 1, BF), lambda e, f: (e * nf + f, 0, 0)),
            pl.BlockSpec((1, D, BF), lambda e, f: (e, 0, f)),
            pl.BlockSpec((1, 1, D), lambda e, f: (e, 0, 0)),
        ],
        out_specs=pl.BlockSpec((1, T, D), lambda e, f: (e, 0, 0)),
        out_shape=jax.ShapeDtypeStruct((E, T, D), jnp.float32),
        compiler_params=pltpu.CompilerParams(
            dimension_semantics=("parallel", "arbitrary")
        ),
    )(xc, cc, W1c, b1r, W2c, b2r)


# speedup vs baseline: 1.4208x; 1.4208x over previous
import functools

import jax
import jax.numpy as jnp
from jax.experimental import pallas as pl
from jax.experimental.pallas import tpu as pltpu


def _ffn_kernel(x_ref, cond_ref, w1_ref, b1_ref, w2_ref, b2_ref, out_ref):
    f = pl.program_id(2)
    a = (x_ref[0] + cond_ref[0]).astype(jnp.bfloat16)
    w1 = w1_ref[0].astype(jnp.bfloat16)
    w2 = w2_ref[0].astype(jnp.bfloat16)
    h = jax.lax.dot_general(
        a, w1,
        dimension_numbers=(((1,), (1,)), ((), ())),
        preferred_element_type=jnp.float32,
    )
    h = jax.nn.gelu(h + b1_ref[0])
    o = jax.lax.dot_general(
        h.astype(jnp.bfloat16), w2,
        dimension_numbers=(((1,), (1,)), ((), ())),
        preferred_element_type=jnp.float32,
    )

    @pl.when(f == 0)
    def _init():
        out_ref[0] = o + b2_ref[0]

    @pl.when(f != 0)
    def _acc():
        out_ref[0] += o


def kernel(x, cond, W1, b1, W2, b2):
    E, T, D = x.shape
    F = W1.shape[1]
    BF = min(512, F)
    nf = F // BF
    BT = min(512, T)
    nt = T // BT

    b1r = b1.reshape(E * nf, 1, BF)
    b2r = b2.reshape(E, 1, D)

    grid = (E, nt, nf)
    return pl.pallas_call(
        _ffn_kernel,
        grid=grid,
        in_specs=[
            pl.BlockSpec((1, BT, D), lambda e, t, f: (e, t, 0)),
            pl.BlockSpec((1, BT, D), lambda e, t, f: (e, t, 0)),
            pl.BlockSpec((1, BF, D), lambda e, t, f: (e, f, 0)),
            pl.BlockSpec((1, 1, BF), lambda e, t, f: (e * nf + f, 0, 0)),
            pl.BlockSpec((1, D, BF), lambda e, t, f: (e, 0, f)),
            pl.BlockSpec((1, 1, D), lambda e, t, f: (e, 0, 0)),
        ],
        out_specs=pl.BlockSpec((1, BT, D), lambda e, t, f: (e, t, 0)),
        out_shape=jax.ShapeDtypeStruct((E, T, D), jnp.float32),
        compiler_params=pltpu.CompilerParams(
            dimension_semantics=("parallel", "parallel", "arbitrary")
        ),
    )(x, cond, W1, b1r, W2, b2r)


# speedup vs baseline: 1.5278x; 1.0752x over previous
import functools

import jax
import jax.numpy as jnp
from jax.experimental import pallas as pl
from jax.experimental.pallas import tpu as pltpu


def _ffn_kernel(x_ref, cond_ref, w1_ref, b1_ref, w2_ref, b2_ref, out_ref):
    f = pl.program_id(1)
    a = x_ref[0] + cond_ref[0]
    w1 = w1_ref[0].astype(jnp.bfloat16)
    w2 = w2_ref[0].astype(jnp.bfloat16)
    h = jax.lax.dot_general(
        a, w1,
        dimension_numbers=(((1,), (1,)), ((), ())),
        preferred_element_type=jnp.float32,
    )
    h = jax.nn.gelu(h + b1_ref[0])
    o = jax.lax.dot_general(
        h.astype(jnp.bfloat16), w2,
        dimension_numbers=(((1,), (1,)), ((), ())),
        preferred_element_type=jnp.float32,
    )

    @pl.when(f == 0)
    def _init():
        out_ref[0] = o + b2_ref[0]

    @pl.when(f != 0)
    def _acc():
        out_ref[0] += o


def kernel(x, cond, W1, b1, W2, b2):
    E, T, D = x.shape
    F = W1.shape[1]
    BF = min(512, F)
    nf = F // BF

    xc = x.astype(jnp.bfloat16)
    cc = cond.astype(jnp.bfloat16)

    b1r = b1.reshape(E * nf, 1, BF)
    b2r = b2.reshape(E, 1, D)

    grid = (E, nf)
    return pl.pallas_call(
        _ffn_kernel,
        grid=grid,
        in_specs=[
            pl.BlockSpec((1, T, D), lambda e, f: (e, 0, 0)),
            pl.BlockSpec((1, T, D), lambda e, f: (e, 0, 0)),
            pl.BlockSpec((1, BF, D), lambda e, f: (e, f, 0)),
            pl.BlockSpec((1, 1, BF), lambda e, f: (e * nf + f, 0, 0)),
            pl.BlockSpec((1, D, BF), lambda e, f: (e, 0, f)),
            pl.BlockSpec((1, 1, D), lambda e, f: (e, 0, 0)),
        ],
        out_specs=pl.BlockSpec((1, T, D), lambda e, f: (e, 0, 0)),
        out_shape=jax.ShapeDtypeStruct((E, T, D), jnp.float32),
        compiler_params=pltpu.CompilerParams(
            dimension_semantics=("parallel", "arbitrary")
        ),
    )(xc, cc, W1, b1r, W2, b2r)
